# Initial kernel scaffold; baseline (speedup 1.0000x reference)
import jax
import jax.numpy as jnp
from jax.experimental import pallas as pl


def kernel(features, points, W1, b1, W2, b2):
    raise NotImplementedError("write your pallas kernel here")


# speedup vs baseline: 29.3038x; 29.3038x over previous
import functools

import jax
import jax.numpy as jnp
from jax.experimental import pallas as pl

_TILE = 256
_K = 8


def _body(pr_ref, pT_ref, fr_ref, fT_ref, W1_ref, b1_ref, W2_ref, b2_ref,
          bp_ref, enh_ref):
    p_row = pr_ref[0]
    pTb = pT_ref[0]
    f_r = fr_ref[0]
    fTb = fT_ref[0]

    dx = p_row[:, 0:1] - pTb[0:1, :]
    dy = p_row[:, 1:2] - pTb[1:2, :]
    dz = p_row[:, 2:3] - pTb[2:3, :]
    d2 = dx * dx + dy * dy + dz * dz

    big = jnp.float32(3e38)
    work = d2
    m = None
    for _ in range(_K):
        m = jnp.min(work, axis=1, keepdims=True)
        work = jnp.where(work <= m, big, work)
    mask = d2 <= m

    fn_all = jnp.sum(fTb * fTb, axis=0, keepdims=True)
    fn_row = jnp.sum(f_r * f_r, axis=1, keepdims=True)
    gram = jax.lax.dot(f_r, fTb, preferred_element_type=jnp.float32)
    fd2 = jnp.maximum(fn_row + (fn_all - 2.0 * gram), 0.0)
    fd = jnp.sqrt(fd2)
    acc = jnp.sum(jnp.where(mask, fd, 0.0), axis=1, keepdims=True)
    fg = acc * (1.0 / _K)

    h = jnp.maximum(
        jax.lax.dot(f_r, W1_ref[...], preferred_element_type=jnp.float32,
                    precision=jax.lax.Precision.HIGHEST) + b1_ref[...], 0.0)
    logits = jax.lax.dot(h, W2_ref[...], preferred_element_type=jnp.float32,
                         precision=jax.lax.Precision.HIGHEST) + b2_ref[...]
    bp = jax.nn.sigmoid(logits)

    enh = f_r + 0.3 * (jnp.tanh(5.0 * fg) * bp)
    bp_ref[0] = bp
    enh_ref[0] = enh


@functools.partial(jax.jit, static_argnames=("interpret",))
def kernel(features, points, W1, b1, W2, b2, interpret=False):
    B, N, D = features.shape
    fT = jnp.swapaxes(features, 1, 2)
    pT = jnp.swapaxes(points, 1, 2)
    b1r = b1.reshape(1, -1)
    W2r = W2.reshape(-1, 1)
    b2r = b2.reshape(1, 1)

    grid = (B, N // _TILE)
    bp, enh = pl.pallas_call(
        _body,
        grid=grid,
        in_specs=[
            pl.BlockSpec((1, _TILE, 3), lambda b, t: (b, t, 0)),
            pl.BlockSpec((1, 3, N), lambda b, t: (b, 0, 0)),
            pl.BlockSpec((1, _TILE, D), lambda b, t: (b, t, 0)),
            pl.BlockSpec((1, D, N), lambda b, t: (b, 0, 0)),
            pl.BlockSpec((D, 64), lambda b, t: (0, 0)),
            pl.BlockSpec((1, 64), lambda b, t: (0, 0)),
            pl.BlockSpec((64, 1), lambda b, t: (0, 0)),
            pl.BlockSpec((1, 1), lambda b, t: (0, 0)),
        ],
        out_specs=[
            pl.BlockSpec((1, _TILE, 1), lambda b, t: (b, t, 0)),
            pl.BlockSpec((1, _TILE, D), lambda b, t: (b, t, 0)),
        ],
        out_shape=[
            jax.ShapeDtypeStruct((B, N, 1), jnp.float32),
            jax.ShapeDtypeStruct((B, N, D), jnp.float32),
        ],
        interpret=interpret,
    )(points, pT, features, fT, W1, b1r, W2r, b2r)
    return (bp, enh)


# speedup vs baseline: 31.9821x; 1.0914x over previous
import functools

import jax
import jax.numpy as jnp
from jax.experimental import pallas as pl

_TILE = 512
_K = 8


def _body(pr_ref, pT_ref, fr_ref, fT_ref, W1_ref, b1_ref, W2_ref, b2_ref,
          bp_ref, enh_ref):
    p_row = pr_ref[0]
    pTb = pT_ref[0]
    f_r = fr_ref[0]
    fTb = fT_ref[0]

    pp = jax.lax.dot(p_row, pTb, preferred_element_type=jnp.float32)
    pn_row = jnp.sum(p_row * p_row, axis=1, keepdims=True)
    pn_all = jnp.sum(pTb * pTb, axis=0, keepdims=True)
    d2 = (pn_row + pn_all) - 2.0 * pp

    big = jnp.float32(3e38)
    work = d2
    m = None
    for k in range(_K):
        m = jnp.min(work, axis=1, keepdims=True)
        if k < _K - 1:
            work = jnp.where(work <= m, big, work)
    mask = d2 <= m

    fn_all = jnp.sum(fTb * fTb, axis=0, keepdims=True)
    fn_row = jnp.sum(f_r * f_r, axis=1, keepdims=True)
    gram = jax.lax.dot(f_r, fTb, preferred_element_type=jnp.float32)
    fd2 = jnp.maximum(fn_row + (fn_all - 2.0 * gram), 0.0)
    fd = jnp.sqrt(fd2)
    acc = jnp.sum(jnp.where(mask, fd, 0.0), axis=1, keepdims=True)
    fg = acc * (1.0 / _K)

    h = jnp.maximum(
        jax.lax.dot(f_r, W1_ref[...], preferred_element_type=jnp.float32,
                    precision=jax.lax.Precision.HIGHEST) + b1_ref[...], 0.0)
    logits = jax.lax.dot(h, W2_ref[...], preferred_element_type=jnp.float32,
                         precision=jax.lax.Precision.HIGHEST) + b2_ref[...]
    bp = jax.nn.sigmoid(logits)

    enh = f_r + 0.3 * (jnp.tanh(5.0 * fg) * bp)
    bp_ref[0] = bp
    enh_ref[0] = enh


@functools.partial(jax.jit, static_argnames=("interpret",))
def kernel(features, points, W1, b1, W2, b2, interpret=False):
    B, N, D = features.shape
    fT = jnp.swapaxes(features, 1, 2)
    pT = jnp.swapaxes(points, 1, 2)
    b1r = b1.reshape(1, -1)
    W2r = W2.reshape(-1, 1)
    b2r = b2.reshape(1, 1)

    grid = (B, N // _TILE)
    bp, enh = pl.pallas_call(
        _body,
        grid=grid,
        in_specs=[
            pl.BlockSpec((1, _TILE, 3), lambda b, t: (b, t, 0)),
            pl.BlockSpec((1, 3, N), lambda b, t: (b, 0, 0)),
            pl.BlockSpec((1, _TILE, D), lambda b, t: (b, t, 0)),
            pl.BlockSpec((1, D, N), lambda b, t: (b, 0, 0)),
            pl.BlockSpec((D, 64), lambda b, t: (0, 0)),
            pl.BlockSpec((1, 64), lambda b, t: (0, 0)),
            pl.BlockSpec((64, 1), lambda b, t: (0, 0)),
            pl.BlockSpec((1, 1), lambda b, t: (0, 0)),
        ],
        out_specs=[
            pl.BlockSpec((1, _TILE, 1), lambda b, t: (b, t, 0)),
            pl.BlockSpec((1, _TILE, D), lambda b, t: (b, t, 0)),
        ],
        out_shape=[
            jax.ShapeDtypeStruct((B, N, 1), jnp.float32),
            jax.ShapeDtypeStruct((B, N, D), jnp.float32),
        ],
        interpret=interpret,
    )(points, pT, features, fT, W1, b1r, W2r, b2r)
    return (bp, enh)


# speedup vs baseline: 33.5711x; 1.0497x over previous
import functools

import jax
import jax.numpy as jnp
from jax.experimental import pallas as pl

_TILE = 512
_K = 8

_DN_T = (((1,), (1,)), ((), ()))


def _body(pr_ref, pa_ref, fr_ref, fa_ref, W1_ref, b1_ref, W2_ref, b2_ref,
          bp_ref, enh_ref):
    p_row = pr_ref[0]
    p_all = pa_ref[0]
    f_r = fr_ref[0]
    f_a = fa_ref[0]

    pp = jax.lax.dot_general(p_row, p_all, _DN_T,
                             preferred_element_type=jnp.float32)
    pn_row = jnp.sum(p_row * p_row, axis=1, keepdims=True)
    pn_all = jnp.sum(p_all * p_all, axis=1, keepdims=True).T
    d2 = (pn_row + pn_all) - 2.0 * pp

    big = jnp.float32(3e38)
    work = d2
    m = None
    for k in range(_K):
        m = jnp.min(work, axis=1, keepdims=True)
        if k < _K - 1:
            work = jnp.where(work <= m, big, work)
    mask = d2 <= m

    fn_all = jnp.sum(f_a * f_a, axis=1, keepdims=True).T
    fn_row = jnp.sum(f_r * f_r, axis=1, keepdims=True)
    gram = jax.lax.dot_general(f_r, f_a, _DN_T,
                               preferred_element_type=jnp.float32)
    fd2 = jnp.maximum(fn_row + (fn_all - 2.0 * gram), 0.0)
    fd = jnp.sqrt(fd2)
    acc = jnp.sum(jnp.where(mask, fd, 0.0), axis=1, keepdims=True)
    fg = acc * (1.0 / _K)

    h = jnp.maximum(
        jax.lax.dot(f_r, W1_ref[...], preferred_element_type=jnp.float32,
                    precision=jax.lax.Precision.HIGHEST) + b1_ref[...], 0.0)
    logits = jax.lax.dot(h, W2_ref[...], preferred_element_type=jnp.float32,
                         precision=jax.lax.Precision.HIGHEST) + b2_ref[...]
    bp = jax.nn.sigmoid(logits)

    enh = f_r + 0.3 * (jnp.tanh(5.0 * fg) * bp)
    bp_ref[0] = bp
    enh_ref[0] = enh


@functools.partial(jax.jit, static_argnames=("interpret",))
def kernel(features, points, W1, b1, W2, b2, interpret=False):
    B, N, D = features.shape
    b1r = b1.reshape(1, -1)
    W2r = W2.reshape(-1, 1)
    b2r = b2.reshape(1, 1)

    grid = (B, N // _TILE)
    bp, enh = pl.pallas_call(
        _body,
        grid=grid,
        in_specs=[
            pl.BlockSpec((1, _TILE, 3), lambda b, t: (b, t, 0)),
            pl.BlockSpec((1, N, 3), lambda b, t: (b, 0, 0)),
            pl.BlockSpec((1, _TILE, D), lambda b, t: (b, t, 0)),
            pl.BlockSpec((1, N, D), lambda b, t: (b, 0, 0)),
            pl.BlockSpec((D, 64), lambda b, t: (0, 0)),
            pl.BlockSpec((1, 64), lambda b, t: (0, 0)),
            pl.BlockSpec((64, 1), lambda b, t: (0, 0)),
            pl.BlockSpec((1, 1), lambda b, t: (0, 0)),
        ],
        out_specs=[
            pl.BlockSpec((1, _TILE, 1), lambda b, t: (b, t, 0)),
            pl.BlockSpec((1, _TILE, D), lambda b, t: (b, t, 0)),
        ],
        out_shape=[
            jax.ShapeDtypeStruct((B, N, 1), jnp.float32),
            jax.ShapeDtypeStruct((B, N, D), jnp.float32),
        ],
        interpret=interpret,
    )(points, points, features, features, W1, b1r, W2r, b2r)
    return (bp, enh)
